# baseline (speedup 1.0000x reference)
import functools

import jax
import jax.numpy as jnp
from jax import lax
from jax.experimental import pallas as pl
from jax.experimental.pallas import tpu as pltpu
from jax.experimental.pallas import tpu_sc as plsc

NN = 100000
EE = 1600000
FF = 30
PP = 32
TT = 3
NC, NS, LL = 2, 16, 16

F32 = jnp.float32
I32 = jnp.int32

CH = 512
NJ = CH // 128
EP = 1605632
ER = EP // 128
NCHUNK = EP // CH
KA = NCHUNK // NS
KS = NCHUNK // (NC * NS)

ACC_ROWS = 100096
RPT = ACC_ROWS // NS
RPT_LAST = NN - (NS - 1) * RPT
DUMMY_ROW = NN

N4 = NN // 4
PL = 128
BN4 = 512
NBLK = -(-N4 // BN4)


def _idx_prep_body(src_ref, et_ref, out_ref):
    base = 2 * (et_ref[...] * NN + src_ref[...])
    out_ref[0] = base
    out_ref[1] = base + 1


def _idx_prep(src2, et2):
    return pl.pallas_call(
        _idx_prep_body,
        out_shape=jax.ShapeDtypeStruct((2, ER, 128), I32),
    )(src2, et2)


def _hcat_body(x_ref, w_ref, b_ref, out_ref):
    x = x_ref[...]
    for i in range(TT):
        out_ref[i] = jnp.dot(x, w_ref[i], preferred_element_type=F32) + b_ref[i]


def _hcat(x4, wT4, bp4):
    return pl.pallas_call(
        _hcat_body,
        grid=(NBLK,),
        in_specs=[
            pl.BlockSpec((BN4, PL), lambda j: (j, 0)),
            pl.BlockSpec((TT, PL, PL), lambda j: (0, 0, 0)),
            pl.BlockSpec((TT, PL), lambda j: (0, 0)),
        ],
        out_specs=pl.BlockSpec((TT, BN4, PL), lambda j: (0, j, 0)),
        out_shape=jax.ShapeDtypeStruct((TT, N4, PL), F32),
    )(x4, wT4, bp4)


def _gru_math(aA_ref, aB_ref, h_ref, wiA_ref, wiB_ref, wh_ref, bi_ref, bh_ref):
    aA = aA_ref[...]
    aB = aB_ref[...]
    h = h_ref[...]
    g = []
    for t in range(TT):
        gi = (jnp.dot(aA, wiA_ref[t], preferred_element_type=F32)
              + jnp.dot(aB, wiB_ref[t], preferred_element_type=F32)
              + bi_ref[t])
        gh = jnp.dot(h, wh_ref[t], preferred_element_type=F32) + bh_ref[t]
        g.append((gi, gh))
    r = jax.nn.sigmoid(g[0][0] + g[0][1])
    z = jax.nn.sigmoid(g[1][0] + g[1][1])
    n = jnp.tanh(g[2][0] + r * g[2][1])
    return (1.0 - z) * n + z * h


def _gru1_body(aA_ref, aB_ref, h_ref, wiA_ref, wiB_ref, wh_ref, bi_ref,
               bh_ref, w2_ref, b2_ref, h1_ref, hc2_ref):
    hr = jnp.maximum(
        _gru_math(aA_ref, aB_ref, h_ref, wiA_ref, wiB_ref, wh_ref,
                  bi_ref, bh_ref), 0.0)
    h1_ref[...] = hr
    for i in range(TT):
        hc2_ref[i] = jnp.dot(hr, w2_ref[i], preferred_element_type=F32) + b2_ref[i]


def _gru2_body(aA_ref, aB_ref, h_ref, wiA_ref, wiB_ref, wh_ref, bi_ref,
               bh_ref, h2_ref):
    h2_ref[...] = _gru_math(aA_ref, aB_ref, h_ref, wiA_ref, wiB_ref, wh_ref,
                            bi_ref, bh_ref)


_GRU_IN_SPECS = [
    pl.BlockSpec((BN4, 64), lambda j: (j, 0)),
    pl.BlockSpec((BN4, 64), lambda j: (j, 0)),
    pl.BlockSpec((BN4, PL), lambda j: (j, 0)),
    pl.BlockSpec((TT, 64, PL), lambda j: (0, 0, 0)),
    pl.BlockSpec((TT, 64, PL), lambda j: (0, 0, 0)),
    pl.BlockSpec((TT, PL, PL), lambda j: (0, 0, 0)),
    pl.BlockSpec((TT, PL), lambda j: (0, 0)),
    pl.BlockSpec((TT, PL), lambda j: (0, 0)),
]


def _gru1(aA, aB, h, wiA, wiB, wh, bi, bh, w2T4, b2p4):
    return pl.pallas_call(
        _gru1_body,
        grid=(NBLK,),
        in_specs=_GRU_IN_SPECS + [
            pl.BlockSpec((TT, PL, PL), lambda j: (0, 0, 0)),
            pl.BlockSpec((TT, PL), lambda j: (0, 0)),
        ],
        out_specs=[
            pl.BlockSpec((BN4, PL), lambda j: (j, 0)),
            pl.BlockSpec((TT, BN4, PL), lambda j: (0, j, 0)),
        ],
        out_shape=[
            jax.ShapeDtypeStruct((N4, PL), F32),
            jax.ShapeDtypeStruct((TT, N4, PL), F32),
        ],
    )(aA, aB, h, wiA, wiB, wh, bi, bh, w2T4, b2p4)


def _gru2(aA, aB, h, wiA, wiB, wh, bi, bh):
    return pl.pallas_call(
        _gru2_body,
        grid=(NBLK,),
        in_specs=_GRU_IN_SPECS,
        out_specs=pl.BlockSpec((BN4, PL), lambda j: (j, 0)),
        out_shape=jax.ShapeDtypeStruct((N4, PL), F32),
    )(aA, aB, h, wiA, wiB, wh, bi, bh)


_SC_MESH = plsc.VectorSubcoreMesh(
    core_axis_name="c", subcore_axis_name="s", num_cores=NC, num_subcores=NS)

_SC_PARAMS = pltpu.CompilerParams(
    use_tc_tiling_on_sc=False, needs_layout_passes=False)


def _sc_scatter_body(hcat_hbm, idx_hbm, dst_hbm, aoutA_hbm, aoutB_hbm,
                     acc, idx_v, dst_v, rows_v, sem_i, sem_g, sem_s):
    cid = lax.axis_index("c")
    sid = lax.axis_index("s")

    def _zfill(i, carry):
        rows_v[i, :] = jnp.zeros((16,), F32)
        return carry
    lax.fori_loop(0, 2 * CH, _zfill, 0)
    for q in range(RPT // (2 * CH)):
        pltpu.sync_copy(rows_v,
                        acc.at[pl.ds(sid * RPT + q * 2 * CH, 2 * CH)])
    pltpu.sync_copy(
        rows_v.at[pl.ds(0, RPT % (2 * CH))],
        acc.at[pl.ds(sid * RPT + (RPT // (2 * CH)) * 2 * CH, RPT % (2 * CH))])
    plsc.subcore_barrier()

    base = sid * KA
    ibase = cid * ER

    def _fire_load(c, b):
        pltpu.async_copy(idx_hbm.at[pl.ds(ibase + c * NJ, NJ)],
                         idx_v.at[b], sem_i)
        pltpu.async_copy(dst_hbm.at[pl.ds(c * NJ, NJ)], dst_v.at[b], sem_i)

    def _wait_load():
        pltpu.make_async_copy(idx_hbm.at[pl.ds(0, NJ)],
                              idx_v.at[0], sem_i).wait()
        pltpu.make_async_copy(dst_hbm.at[pl.ds(0, NJ)],
                              dst_v.at[0], sem_i).wait()

    def _fire_gathers(b, bo):
        for j in range(NJ):
            pltpu.async_copy(hcat_hbm.at[idx_v.at[b, j]],
                             rows_v.at[pl.ds(bo + j * 128, 128)], sem_g)

    def _wait_gathers():
        for j in range(NJ):
            pltpu.make_async_copy(hcat_hbm.at[idx_v.at[0, 0]],
                                  rows_v.at[pl.ds(j * 128, 128)],
                                  sem_g).wait()

    def _fire_scatters(b3, bo):
        for j in range(NJ):
            pltpu.async_copy(rows_v.at[pl.ds(bo + j * 128, 128)],
                             acc.at[dst_v.at[b3, j]], sem_s, add=True)

    def _wait_scatters():
        for j in range(NJ):
            pltpu.make_async_copy(rows_v.at[pl.ds(j * 128, 128)],
                                  acc.at[dst_v.at[0, 0]], sem_s).wait()

    pltpu.sync_copy(idx_hbm.at[pl.ds(ibase + base * NJ, NJ)], idx_v.at[0])
    pltpu.sync_copy(dst_hbm.at[pl.ds(base * NJ, NJ)], dst_v.at[0])
    _fire_gathers(0, 0)
    _fire_load(base + 1, 1)

    def _step(k, carry):
        bo = lax.rem(k, 2) * CH
        nbo = lax.rem(k + 1, 2) * CH
        b3 = lax.rem(k, 3)
        nb3 = lax.rem(k + 1, 3)
        c2 = base + jnp.minimum(k + 2, KA - 1)
        _wait_gathers()
        _fire_scatters(b3, bo)

        @pl.when(k > 0)
        def _():
            _wait_scatters()
        _fire_load(c2, lax.rem(k + 2, 3))
        _wait_load()
        _fire_gathers(nb3, nbo)
        return carry
    lax.fori_loop(0, KA, _step, 0)
    _wait_scatters()
    _wait_gathers()
    _wait_load()
    plsc.subcore_barrier()

    for c, aout_hbm in ((0, aoutA_hbm), (1, aoutB_hbm)):
        @pl.when(jnp.logical_and(cid == c, sid < NS - 1))
        def _copy_full(aout_hbm=aout_hbm):
            pltpu.sync_copy(
                acc.at[pl.ds(sid * RPT, RPT)],
                aout_hbm.at[pl.ds(sid * RPT, RPT)])

        @pl.when(jnp.logical_and(cid == c, sid == NS - 1))
        def _copy_last(aout_hbm=aout_hbm):
            pltpu.sync_copy(
                acc.at[pl.ds((NS - 1) * RPT, RPT_LAST)],
                aout_hbm.at[pl.ds((NS - 1) * RPT, RPT_LAST)])


_sc_scatter = pl.kernel(
    _sc_scatter_body,
    out_type=(jax.ShapeDtypeStruct((NN, 16), F32),
              jax.ShapeDtypeStruct((NN, 16), F32)),
    mesh=_SC_MESH,
    compiler_params=_SC_PARAMS,
    scratch_types=[
        pltpu.VMEM_SHARED((ACC_ROWS, 16), F32),
        pltpu.VMEM((3, NJ, 128), I32),
        pltpu.VMEM((3, NJ, 128), I32),
        pltpu.VMEM((2 * CH, 16), F32),
        pltpu.SemaphoreType.DMA,
        pltpu.SemaphoreType.DMA,
        pltpu.SemaphoreType.DMA,
    ],
)


KS0 = 88
KS1 = 2 * KS - KS0


def _sc_score_body(h2_hbm, ei_hbm, nei_hbm, pos_hbm, neg_hbm,
                   sidx_v, didx_v, u_v, v_v, s_v, sem_i, sem_g, sem_o):
    cid = lax.axis_index("c")
    sid = lax.axis_index("s")
    base = jnp.where(cid == 0, sid * KS0, NS * KS0 + sid * KS1)
    count = jnp.where(cid == 0, KS0, KS1)
    lane = lax.iota(I32, 16)

    for earr, oarr in ((ei_hbm, pos_hbm), (nei_hbm, neg_hbm)):
        def _fire_load(c, b, earr=earr):
            pltpu.async_copy(earr.at[0, pl.ds(c * NJ, NJ)],
                             sidx_v.at[b], sem_i)
            pltpu.async_copy(earr.at[1, pl.ds(c * NJ, NJ)],
                             didx_v.at[b], sem_i)

        def _wait_load(earr=earr):
            pltpu.make_async_copy(earr.at[0, pl.ds(0, NJ)],
                                  sidx_v.at[0], sem_i).wait()
            pltpu.make_async_copy(earr.at[1, pl.ds(0, NJ)],
                                  didx_v.at[0], sem_i).wait()

        def _fire_gathers(b, bo):
            for j in range(NJ):
                pltpu.async_copy(h2_hbm.at[sidx_v.at[b, j]],
                                 u_v.at[pl.ds(bo + j * 128, 128)], sem_g)
                pltpu.async_copy(h2_hbm.at[didx_v.at[b, j]],
                                 v_v.at[pl.ds(bo + j * 128, 128)], sem_g)

        def _wait_gathers():
            for j in range(NJ):
                pltpu.make_async_copy(h2_hbm.at[sidx_v.at[0, 0]],
                                      u_v.at[pl.ds(j * 128, 128)],
                                      sem_g).wait()
                pltpu.make_async_copy(h2_hbm.at[sidx_v.at[0, 0]],
                                      v_v.at[pl.ds(j * 128, 128)],
                                      sem_g).wait()

        def _wait_store(oarr=oarr):
            pltpu.make_async_copy(s_v.at[0], oarr.at[pl.ds(0, CH)],
                                  sem_o).wait()

        pltpu.sync_copy(earr.at[0, pl.ds(base * NJ, NJ)], sidx_v.at[0])
        pltpu.sync_copy(earr.at[1, pl.ds(base * NJ, NJ)], didx_v.at[0])
        _fire_gathers(0, 0)
        _fire_load(base + 1, 1)

        def _step(k, carry):
            b = lax.rem(k, 2)
            bo = b * CH
            nb = lax.rem(k + 1, 2)
            nbo = nb * CH
            c = base + k
            c2 = base + jnp.minimum(k + 2, count - 1)
            _wait_gathers()
            _fire_load(c2, b)
            _wait_load()
            _fire_gathers(nb, nbo)

            @pl.when(k > 1)
            def _():
                _wait_store()

            def _grp(g, c3):
                rows = bo + g * 16 + lane
                acc16 = jnp.zeros((16,), F32)
                for f in range(PP):
                    cols = jnp.bitwise_and(f + lane, PP - 1)
                    u = plsc.load_gather(u_v, [rows, cols])
                    v = plsc.load_gather(v_v, [rows, cols])
                    acc16 = acc16 + u * v
                s_v[b, pl.ds(g * 16, 16)] = acc16
                return c3
            lax.fori_loop(0, CH // 16, _grp, 0)
            pltpu.async_copy(s_v.at[b], oarr.at[pl.ds(c * CH, CH)], sem_o)
            return carry
        lax.fori_loop(0, count, _step, 0)
        _wait_gathers()
        _wait_load()
        _wait_store()
        _wait_store()


_sc_score = pl.kernel(
    _sc_score_body,
    out_type=(jax.ShapeDtypeStruct((EP,), F32),
              jax.ShapeDtypeStruct((EP,), F32)),
    mesh=_SC_MESH,
    compiler_params=_SC_PARAMS,
    scratch_types=[
        pltpu.VMEM((2, NJ, 128), I32),
        pltpu.VMEM((2, NJ, 128), I32),
        pltpu.VMEM((2 * CH, PP), F32),
        pltpu.VMEM((2 * CH, PP), F32),
        pltpu.VMEM((2, CH), F32),
        pltpu.SemaphoreType.DMA,
        pltpu.SemaphoreType.DMA,
        pltpu.SemaphoreType.DMA,
    ],
)


def _kron4(w):
    return jnp.kron(jnp.eye(4, dtype=F32), w)


def _pad_wT(W):
    return jnp.pad(jnp.transpose(W, (0, 2, 1)),
                   ((0, 0), (0, PP - FF), (0, PP - FF)))


def _pack_w4(W):
    wT = _pad_wT(W)
    return jnp.stack([_kron4(wT[i]) for i in range(TT)])


def _pack_b4(b):
    return jnp.tile(jnp.pad(b, ((0, 0), (0, PP - FF))), (1, 4))


def _pack_gru4(wih, whh, bih, bhh):
    w3 = jnp.pad(jnp.transpose(wih.reshape(3, FF, FF), (0, 2, 1)),
                 ((0, 0), (0, PP - FF), (0, PP - FF)))
    wiA = jnp.stack([_kron4(w3[t][0:16, :]) for t in range(TT)])
    wiB = jnp.stack([_kron4(w3[t][16:32, :]) for t in range(TT)])
    wh3 = jnp.pad(jnp.transpose(whh.reshape(3, FF, FF), (0, 2, 1)),
                  ((0, 0), (0, PP - FF), (0, PP - FF)))
    wh = jnp.stack([_kron4(wh3[t]) for t in range(TT)])
    bi = _pack_b4(bih.reshape(3, FF))
    bh = _pack_b4(bhh.reshape(3, FF))
    return wiA, wiB, wh, bi, bh


def kernel(inputs, W1, b1, g1_wih, g1_whh, g1_bih, g1_bhh,
           W2, b2, g2_wih, g2_whh, g2_bih, g2_bhh,
           edge_index, edge_types, neg_edge_index):
    xp = jnp.pad(inputs, ((0, 0), (0, PP - FF)))
    x4 = xp.reshape(N4, PL)
    w1T4 = _pack_w4(W1)
    b1p4 = _pack_b4(b1)
    w2T4 = _pack_w4(W2)
    b2p4 = _pack_b4(b2)
    wiA1, wiB1, wh1, bi1, bh1 = _pack_gru4(g1_wih, g1_whh, g1_bih, g1_bhh)
    wiA2, wiB2, wh2, bi2, bh2 = _pack_gru4(g2_wih, g2_whh, g2_bih, g2_bhh)

    pad_e = EP - EE
    src2 = jnp.pad(edge_index[0], (0, pad_e)).reshape(ER, 128)
    et2 = jnp.pad(edge_types, (0, pad_e)).reshape(ER, 128)
    dst2 = jnp.pad(edge_index[1], (0, pad_e),
                   constant_values=DUMMY_ROW).reshape(ER, 128)
    ei3 = jnp.pad(edge_index, ((0, 0), (0, pad_e))).reshape(2, ER, 128)
    nei3 = jnp.pad(neg_edge_index, ((0, 0), (0, pad_e))).reshape(2, ER, 128)

    idx = _idx_prep(src2, et2).reshape(2 * ER, 128)

    hcat1 = _hcat(x4, w1T4, b1p4).reshape(2 * TT * NN, 16)
    a1A, a1B = _sc_scatter(hcat1, idx, dst2)
    h1, hcat2 = _gru1(a1A.reshape(N4, 64), a1B.reshape(N4, 64), x4,
                      wiA1, wiB1, wh1, bi1, bh1, w2T4, b2p4)
    a2A, a2B = _sc_scatter(hcat2.reshape(2 * TT * NN, 16), idx, dst2)
    h2 = _gru2(a2A.reshape(N4, 64), a2B.reshape(N4, 64), h1,
               wiA2, wiB2, wh2, bi2, bh2)

    pos, neg = _sc_score(h2.reshape(NN, PP), ei3, nei3)
    return (pos[:EE].reshape(EE, 1), neg[:EE].reshape(EE, 1))


# speedup vs baseline: 1.0357x; 1.0357x over previous
import functools

import jax
import jax.numpy as jnp
from jax import lax
from jax.experimental import pallas as pl
from jax.experimental.pallas import tpu as pltpu
from jax.experimental.pallas import tpu_sc as plsc

NN = 100000
EE = 1600000
FF = 30
PP = 32
TT = 3
NC, NS, LL = 2, 16, 16

F32 = jnp.float32
I32 = jnp.int32

CH = 512
NJ = CH // 128
EP = 1605632
ER = EP // 128
NCHUNK = EP // CH
KA = NCHUNK // NS
KS = NCHUNK // (NC * NS)

ACC_ROWS = 100096
RPT = ACC_ROWS // NS
RPT_LAST = NN - (NS - 1) * RPT
DUMMY_ROW = NN

N4 = NN // 4
PL = 128
BN4 = 512
NBLK = -(-N4 // BN4)


def _idx_prep_body(src_ref, et_ref, out_ref):
    base = 2 * (et_ref[...] * NN + src_ref[...])
    out_ref[0] = base
    out_ref[1] = base + 1


def _idx_prep(src2, et2):
    return pl.pallas_call(
        _idx_prep_body,
        out_shape=jax.ShapeDtypeStruct((2, ER, 128), I32),
    )(src2, et2)


def _hcat_body(x_ref, w_ref, b_ref, out_ref):
    x = x_ref[...]
    for i in range(TT):
        out_ref[i] = jnp.dot(x, w_ref[i], preferred_element_type=F32) + b_ref[i]


def _hcat(x4, wT4, bp4):
    return pl.pallas_call(
        _hcat_body,
        grid=(NBLK,),
        in_specs=[
            pl.BlockSpec((BN4, PL), lambda j: (j, 0)),
            pl.BlockSpec((TT, PL, PL), lambda j: (0, 0, 0)),
            pl.BlockSpec((TT, PL), lambda j: (0, 0)),
        ],
        out_specs=pl.BlockSpec((TT, BN4, PL), lambda j: (0, j, 0)),
        out_shape=jax.ShapeDtypeStruct((TT, N4, PL), F32),
    )(x4, wT4, bp4)


def _gru_math(aA_ref, aB_ref, h_ref, wiA_ref, wiB_ref, wh_ref, bi_ref, bh_ref):
    aA = aA_ref[...]
    aB = aB_ref[...]
    h = h_ref[...]
    g = []
    for t in range(TT):
        gi = (jnp.dot(aA, wiA_ref[t], preferred_element_type=F32)
              + jnp.dot(aB, wiB_ref[t], preferred_element_type=F32)
              + bi_ref[t])
        gh = jnp.dot(h, wh_ref[t], preferred_element_type=F32) + bh_ref[t]
        g.append((gi, gh))
    r = jax.nn.sigmoid(g[0][0] + g[0][1])
    z = jax.nn.sigmoid(g[1][0] + g[1][1])
    n = jnp.tanh(g[2][0] + r * g[2][1])
    return (1.0 - z) * n + z * h


def _gru1_body(aA_ref, aB_ref, h_ref, wiA_ref, wiB_ref, wh_ref, bi_ref,
               bh_ref, w2_ref, b2_ref, h1_ref, hc2_ref):
    hr = jnp.maximum(
        _gru_math(aA_ref, aB_ref, h_ref, wiA_ref, wiB_ref, wh_ref,
                  bi_ref, bh_ref), 0.0)
    h1_ref[...] = hr
    for i in range(TT):
        hc2_ref[i] = jnp.dot(hr, w2_ref[i], preferred_element_type=F32) + b2_ref[i]


def _gru2_body(aA_ref, aB_ref, h_ref, wiA_ref, wiB_ref, wh_ref, bi_ref,
               bh_ref, h2_ref):
    h2_ref[...] = _gru_math(aA_ref, aB_ref, h_ref, wiA_ref, wiB_ref, wh_ref,
                            bi_ref, bh_ref)


_GRU_IN_SPECS = [
    pl.BlockSpec((BN4, 64), lambda j: (j, 0)),
    pl.BlockSpec((BN4, 64), lambda j: (j, 0)),
    pl.BlockSpec((BN4, PL), lambda j: (j, 0)),
    pl.BlockSpec((TT, 64, PL), lambda j: (0, 0, 0)),
    pl.BlockSpec((TT, 64, PL), lambda j: (0, 0, 0)),
    pl.BlockSpec((TT, PL, PL), lambda j: (0, 0, 0)),
    pl.BlockSpec((TT, PL), lambda j: (0, 0)),
    pl.BlockSpec((TT, PL), lambda j: (0, 0)),
]


def _gru1(aA, aB, h, wiA, wiB, wh, bi, bh, w2T4, b2p4):
    return pl.pallas_call(
        _gru1_body,
        grid=(NBLK,),
        in_specs=_GRU_IN_SPECS + [
            pl.BlockSpec((TT, PL, PL), lambda j: (0, 0, 0)),
            pl.BlockSpec((TT, PL), lambda j: (0, 0)),
        ],
        out_specs=[
            pl.BlockSpec((BN4, PL), lambda j: (j, 0)),
            pl.BlockSpec((TT, BN4, PL), lambda j: (0, j, 0)),
        ],
        out_shape=[
            jax.ShapeDtypeStruct((N4, PL), F32),
            jax.ShapeDtypeStruct((TT, N4, PL), F32),
        ],
    )(aA, aB, h, wiA, wiB, wh, bi, bh, w2T4, b2p4)


def _gru2(aA, aB, h, wiA, wiB, wh, bi, bh):
    return pl.pallas_call(
        _gru2_body,
        grid=(NBLK,),
        in_specs=_GRU_IN_SPECS,
        out_specs=pl.BlockSpec((BN4, PL), lambda j: (j, 0)),
        out_shape=jax.ShapeDtypeStruct((N4, PL), F32),
    )(aA, aB, h, wiA, wiB, wh, bi, bh)


_SC_MESH = plsc.VectorSubcoreMesh(
    core_axis_name="c", subcore_axis_name="s", num_cores=NC, num_subcores=NS)

_SC_PARAMS = pltpu.CompilerParams(
    use_tc_tiling_on_sc=False, needs_layout_passes=False)


def _sc_scatter_body(hcat_hbm, idx_hbm, dst_hbm, aoutA_hbm, aoutB_hbm,
                     acc, idx_v, dst_v, rows_v, sem_i, sem_g, sem_s):
    cid = lax.axis_index("c")
    sid = lax.axis_index("s")

    def _zfill(i, carry):
        rows_v[i, :] = jnp.zeros((16,), F32)
        return carry
    lax.fori_loop(0, 2 * CH, _zfill, 0)
    for q in range(RPT // (2 * CH)):
        pltpu.sync_copy(rows_v,
                        acc.at[pl.ds(sid * RPT + q * 2 * CH, 2 * CH)])
    pltpu.sync_copy(
        rows_v.at[pl.ds(0, RPT % (2 * CH))],
        acc.at[pl.ds(sid * RPT + (RPT // (2 * CH)) * 2 * CH, RPT % (2 * CH))])
    plsc.subcore_barrier()

    base = sid * KA
    ibase = cid * ER

    def _fire_load(c, b):
        pltpu.async_copy(idx_hbm.at[pl.ds(ibase + c * NJ, NJ)],
                         idx_v.at[b], sem_i)
        pltpu.async_copy(dst_hbm.at[pl.ds(c * NJ, NJ)], dst_v.at[b], sem_i)

    def _wait_load():
        pltpu.make_async_copy(idx_hbm.at[pl.ds(0, NJ)],
                              idx_v.at[0], sem_i).wait()
        pltpu.make_async_copy(dst_hbm.at[pl.ds(0, NJ)],
                              dst_v.at[0], sem_i).wait()

    def _fire_gathers(b, bo):
        for j in range(NJ):
            pltpu.async_copy(hcat_hbm.at[idx_v.at[b, j]],
                             rows_v.at[pl.ds(bo + j * 128, 128)], sem_g)

    def _wait_gathers():
        for j in range(NJ):
            pltpu.make_async_copy(hcat_hbm.at[idx_v.at[0, 0]],
                                  rows_v.at[pl.ds(j * 128, 128)],
                                  sem_g).wait()

    def _fire_scatters(b3, bo):
        for j in range(NJ):
            pltpu.async_copy(rows_v.at[pl.ds(bo + j * 128, 128)],
                             acc.at[dst_v.at[b3, j]], sem_s, add=True)

    def _wait_scatters():
        for j in range(NJ):
            pltpu.make_async_copy(rows_v.at[pl.ds(j * 128, 128)],
                                  acc.at[dst_v.at[0, 0]], sem_s).wait()

    pltpu.sync_copy(idx_hbm.at[pl.ds(ibase + base * NJ, NJ)], idx_v.at[0])
    pltpu.sync_copy(dst_hbm.at[pl.ds(base * NJ, NJ)], dst_v.at[0])
    _fire_gathers(0, 0)
    _fire_load(base + 1, 1)

    def _step(k, carry):
        bo = lax.rem(k, 2) * CH
        nbo = lax.rem(k + 1, 2) * CH
        b3 = lax.rem(k, 3)
        nb3 = lax.rem(k + 1, 3)
        c2 = base + jnp.minimum(k + 2, KA - 1)
        _wait_gathers()
        _fire_scatters(b3, bo)

        @pl.when(k > 0)
        def _():
            _wait_scatters()
        _fire_load(c2, lax.rem(k + 2, 3))
        _wait_load()
        _fire_gathers(nb3, nbo)
        return carry
    lax.fori_loop(0, KA, _step, 0)
    _wait_scatters()
    _wait_gathers()
    _wait_load()
    plsc.subcore_barrier()

    for c, aout_hbm in ((0, aoutA_hbm), (1, aoutB_hbm)):
        @pl.when(jnp.logical_and(cid == c, sid < NS - 1))
        def _copy_full(aout_hbm=aout_hbm):
            pltpu.sync_copy(
                acc.at[pl.ds(sid * RPT, RPT)],
                aout_hbm.at[pl.ds(sid * RPT, RPT)])

        @pl.when(jnp.logical_and(cid == c, sid == NS - 1))
        def _copy_last(aout_hbm=aout_hbm):
            pltpu.sync_copy(
                acc.at[pl.ds((NS - 1) * RPT, RPT_LAST)],
                aout_hbm.at[pl.ds((NS - 1) * RPT, RPT_LAST)])


_sc_scatter = pl.kernel(
    _sc_scatter_body,
    out_type=(jax.ShapeDtypeStruct((NN, 16), F32),
              jax.ShapeDtypeStruct((NN, 16), F32)),
    mesh=_SC_MESH,
    compiler_params=_SC_PARAMS,
    scratch_types=[
        pltpu.VMEM_SHARED((ACC_ROWS, 16), F32),
        pltpu.VMEM((3, NJ, 128), I32),
        pltpu.VMEM((3, NJ, 128), I32),
        pltpu.VMEM((2 * CH, 16), F32),
        pltpu.SemaphoreType.DMA,
        pltpu.SemaphoreType.DMA,
        pltpu.SemaphoreType.DMA,
    ],
)


KS0 = 108
KS1 = 2 * KS - KS0


def _sc_score_body(h2_hbm, ei_hbm, nei_hbm, pos_hbm, neg_hbm,
                   sidx_v, didx_v, u_v, v_v, s_v, sem_i, sem_g, sem_o):
    cid = lax.axis_index("c")
    sid = lax.axis_index("s")
    base = jnp.where(cid == 0, sid * KS0, NS * KS0 + sid * KS1)
    count = jnp.where(cid == 0, KS0, KS1)
    lane = lax.iota(I32, 16)

    for earr, oarr in ((ei_hbm, pos_hbm), (nei_hbm, neg_hbm)):
        def _fire_load(c, b, earr=earr):
            pltpu.async_copy(earr.at[0, pl.ds(c * NJ, NJ)],
                             sidx_v.at[b], sem_i)
            pltpu.async_copy(earr.at[1, pl.ds(c * NJ, NJ)],
                             didx_v.at[b], sem_i)

        def _wait_load(earr=earr):
            pltpu.make_async_copy(earr.at[0, pl.ds(0, NJ)],
                                  sidx_v.at[0], sem_i).wait()
            pltpu.make_async_copy(earr.at[1, pl.ds(0, NJ)],
                                  didx_v.at[0], sem_i).wait()

        def _fire_gathers(b, bo):
            for j in range(NJ):
                pltpu.async_copy(h2_hbm.at[sidx_v.at[b, j]],
                                 u_v.at[pl.ds(bo + j * 128, 128)], sem_g)
                pltpu.async_copy(h2_hbm.at[didx_v.at[b, j]],
                                 v_v.at[pl.ds(bo + j * 128, 128)], sem_g)

        def _wait_gathers():
            for j in range(NJ):
                pltpu.make_async_copy(h2_hbm.at[sidx_v.at[0, 0]],
                                      u_v.at[pl.ds(j * 128, 128)],
                                      sem_g).wait()
                pltpu.make_async_copy(h2_hbm.at[sidx_v.at[0, 0]],
                                      v_v.at[pl.ds(j * 128, 128)],
                                      sem_g).wait()

        def _wait_store(oarr=oarr):
            pltpu.make_async_copy(s_v.at[0], oarr.at[pl.ds(0, CH)],
                                  sem_o).wait()

        pltpu.sync_copy(earr.at[0, pl.ds(base * NJ, NJ)], sidx_v.at[0])
        pltpu.sync_copy(earr.at[1, pl.ds(base * NJ, NJ)], didx_v.at[0])
        _fire_gathers(0, 0)
        _fire_load(base + 1, 1)

        def _step(k, carry):
            b = lax.rem(k, 2)
            bo = b * CH
            nb = lax.rem(k + 1, 2)
            nbo = nb * CH
            c = base + k
            c2 = base + jnp.minimum(k + 2, count - 1)
            _wait_gathers()
            _fire_load(c2, b)
            _wait_load()
            _fire_gathers(nb, nbo)

            @pl.when(k > 1)
            def _():
                _wait_store()

            def _grp(g, c3):
                rows = bo + g * 16 + lane
                acc16 = jnp.zeros((16,), F32)
                for f in range(PP):
                    cols = jnp.bitwise_and(f + lane, PP - 1)
                    u = plsc.load_gather(u_v, [rows, cols])
                    v = plsc.load_gather(v_v, [rows, cols])
                    acc16 = acc16 + u * v
                s_v[b, pl.ds(g * 16, 16)] = acc16
                return c3
            lax.fori_loop(0, CH // 16, _grp, 0)
            pltpu.async_copy(s_v.at[b], oarr.at[pl.ds(c * CH, CH)], sem_o)
            return carry
        lax.fori_loop(0, count, _step, 0)
        _wait_gathers()
        _wait_load()
        _wait_store()
        _wait_store()


_sc_score = pl.kernel(
    _sc_score_body,
    out_type=(jax.ShapeDtypeStruct((EP,), F32),
              jax.ShapeDtypeStruct((EP,), F32)),
    mesh=_SC_MESH,
    compiler_params=_SC_PARAMS,
    scratch_types=[
        pltpu.VMEM((2, NJ, 128), I32),
        pltpu.VMEM((2, NJ, 128), I32),
        pltpu.VMEM((2 * CH, PP), F32),
        pltpu.VMEM((2 * CH, PP), F32),
        pltpu.VMEM((2, CH), F32),
        pltpu.SemaphoreType.DMA,
        pltpu.SemaphoreType.DMA,
        pltpu.SemaphoreType.DMA,
    ],
)


def _kron4(w):
    return jnp.kron(jnp.eye(4, dtype=F32), w)


def _pad_wT(W):
    return jnp.pad(jnp.transpose(W, (0, 2, 1)),
                   ((0, 0), (0, PP - FF), (0, PP - FF)))


def _pack_w4(W):
    wT = _pad_wT(W)
    return jnp.stack([_kron4(wT[i]) for i in range(TT)])


def _pack_b4(b):
    return jnp.tile(jnp.pad(b, ((0, 0), (0, PP - FF))), (1, 4))


def _pack_gru4(wih, whh, bih, bhh):
    w3 = jnp.pad(jnp.transpose(wih.reshape(3, FF, FF), (0, 2, 1)),
                 ((0, 0), (0, PP - FF), (0, PP - FF)))
    wiA = jnp.stack([_kron4(w3[t][0:16, :]) for t in range(TT)])
    wiB = jnp.stack([_kron4(w3[t][16:32, :]) for t in range(TT)])
    wh3 = jnp.pad(jnp.transpose(whh.reshape(3, FF, FF), (0, 2, 1)),
                  ((0, 0), (0, PP - FF), (0, PP - FF)))
    wh = jnp.stack([_kron4(wh3[t]) for t in range(TT)])
    bi = _pack_b4(bih.reshape(3, FF))
    bh = _pack_b4(bhh.reshape(3, FF))
    return wiA, wiB, wh, bi, bh


def kernel(inputs, W1, b1, g1_wih, g1_whh, g1_bih, g1_bhh,
           W2, b2, g2_wih, g2_whh, g2_bih, g2_bhh,
           edge_index, edge_types, neg_edge_index):
    xp = jnp.pad(inputs, ((0, 0), (0, PP - FF)))
    x4 = xp.reshape(N4, PL)
    w1T4 = _pack_w4(W1)
    b1p4 = _pack_b4(b1)
    w2T4 = _pack_w4(W2)
    b2p4 = _pack_b4(b2)
    wiA1, wiB1, wh1, bi1, bh1 = _pack_gru4(g1_wih, g1_whh, g1_bih, g1_bhh)
    wiA2, wiB2, wh2, bi2, bh2 = _pack_gru4(g2_wih, g2_whh, g2_bih, g2_bhh)

    pad_e = EP - EE
    src2 = jnp.pad(edge_index[0], (0, pad_e)).reshape(ER, 128)
    et2 = jnp.pad(edge_types, (0, pad_e)).reshape(ER, 128)
    dst2 = jnp.pad(edge_index[1], (0, pad_e),
                   constant_values=DUMMY_ROW).reshape(ER, 128)
    ei3 = jnp.pad(edge_index, ((0, 0), (0, pad_e))).reshape(2, ER, 128)
    nei3 = jnp.pad(neg_edge_index, ((0, 0), (0, pad_e))).reshape(2, ER, 128)

    idx = _idx_prep(src2, et2).reshape(2 * ER, 128)

    hcat1 = _hcat(x4, w1T4, b1p4).reshape(2 * TT * NN, 16)
    a1A, a1B = _sc_scatter(hcat1, idx, dst2)
    h1, hcat2 = _gru1(a1A.reshape(N4, 64), a1B.reshape(N4, 64), x4,
                      wiA1, wiB1, wh1, bi1, bh1, w2T4, b2p4)
    a2A, a2B = _sc_scatter(hcat2.reshape(2 * TT * NN, 16), idx, dst2)
    h2 = _gru2(a2A.reshape(N4, 64), a2B.reshape(N4, 64), h1,
               wiA2, wiB2, wh2, bi2, bh2)

    pos, neg = _sc_score(h2.reshape(NN, PP), ei3, nei3)
    return (pos[:EE].reshape(EE, 1), neg[:EE].reshape(EE, 1))
